# baseline (speedup 1.0000x reference)
import jax
import jax.numpy as jnp
from jax import lax
from jax.experimental import pallas as pl
from jax.experimental.pallas import tpu as pltpu
from jax.experimental.pallas import tpu_sc as plsc

_M, _L = 8192, 2048
_SCALE = 0.1
_N = (_L * (_L + 1)) // 2
_NW = 32
_WPAD = _L + 128
_TOPW = _L // _NW
_BC = 8
_NCH = (_M - _L) // _NW // _BC
_CHUNKS = _L // 16
_UNROLL = 8


def _sc_body(corr, base, out,
             win_a, win_b, base_a, base_b, out_a, out_b,
             wsem, bsem, osem):
    wid = lax.axis_index("c") * 16 + lax.axis_index("s")


    def row_params(k):
        m = wid + k * _NW
        start = m * _L - (m * (m + 1)) // 2
        start8 = jnp.minimum((start // 8) * 8, _N - _WPAD)
        start8 = pl.multiple_of(start8, 8)
        return m, start8, start - start8

    def win_copy(k, buf, slot):
        _, start8, _ = row_params(k)
        return pltpu.make_async_copy(
            corr.at[pl.ds(start8, _WPAD)], buf, wsem.at[slot])

    def base_copy(k, buf, slot):
        m, _, _ = row_params(k)
        return pltpu.make_async_copy(base.at[m], buf, bsem.at[slot])

    def out_copy(k, buf, slot):
        m, _, _ = row_params(k)
        return pltpu.make_async_copy(buf, out.at[m], osem.at[slot])

    def compute_row(k, wbuf, bbuf, obuf):
        m, _, d = row_params(k)
        lane = lax.broadcasted_iota(jnp.int32, (16,), 0)
        dr = d & 15
        dq16 = pl.multiple_of(d - dr, 16)
        ilo = (dr + lane) & 15
        from_lo = (dr + lane) < 16

        def window(j):
            c0 = pl.multiple_of(j * 16, 16)
            off = pl.multiple_of(dq16 + c0, 16)
            off_hi = pl.multiple_of(jnp.minimum(off + 16, _WPAD - 16), 16)
            x_lo = wbuf[pl.ds(off, 16)]
            x_hi = wbuf[pl.ds(off_hi, 16)]
            return jnp.where(
                from_lo,
                x_lo.at[ilo].get(mode="promise_in_bounds"),
                x_hi.at[ilo].get(mode="promise_in_bounds"),
            )

        def copy1(j):
            c0 = pl.multiple_of(j * 16, 16)
            obuf[pl.ds(c0, 16)] = bbuf[pl.ds(c0, 16)]

        def fma1(j):
            c0 = pl.multiple_of(j * 16, 16)
            obuf[pl.ds(c0, 16)] = bbuf[pl.ds(c0, 16)] + _SCALE * window(j)

        jb = m >> 4
        jb8 = jb >> 3
        ju = jb + 1
        ju8 = ((ju + 7) >> 3) << 3

        def copy8(s, c):
            for t in range(_UNROLL):
                copy1(s * _UNROLL + t)
            return c

        def copy_tail(j, c):
            copy1(j)
            return c

        def fma_head(j, c):
            fma1(j)
            return c

        def fma8(s, c):
            for t in range(_UNROLL):
                fma1(s * _UNROLL + t)
            return c

        lax.fori_loop(0, jb8, copy8, 0)
        lax.fori_loop(jb8 * _UNROLL, jb, copy_tail, 0)

        c0 = pl.multiple_of(jb * 16, 16)
        keep = (c0 + lane) >= m
        obuf[pl.ds(c0, 16)] = bbuf[pl.ds(c0, 16)] + jnp.where(
            keep, _SCALE * window(jb), 0.0)

        lax.fori_loop(ju, ju8, fma_head, 0)
        lax.fori_loop(ju8 >> 3, _CHUNKS // _UNROLL, fma8, 0)

    win_copy(0, win_a, 0).start()
    base_copy(0, base_a, 0).start()

    def top_body(k2, carry):
        k = 2 * k2
        win_copy(k + 1, win_b, 1).start()
        base_copy(k + 1, base_b, 1).start()
        win_copy(k, win_a, 0).wait()
        base_copy(k, base_a, 0).wait()

        @pl.when(k2 >= 1)
        def _drain_a():
            out_copy(k - 2, out_a, 0).wait()

        compute_row(k, win_a, base_a, out_a)
        out_copy(k, out_a, 0).start()

        @pl.when(k + 2 < _TOPW)
        def _prefetch_a():
            win_copy(k + 2, win_a, 0).start()
            base_copy(k + 2, base_a, 0).start()

        win_copy(k + 1, win_b, 1).wait()
        base_copy(k + 1, base_b, 1).wait()

        @pl.when(k2 >= 1)
        def _drain_b():
            out_copy(k - 1, out_b, 1).wait()

        compute_row(k + 1, win_b, base_b, out_b)
        out_copy(k + 1, out_b, 1).start()
        return carry

    lax.fori_loop(0, _TOPW // 2, top_body, 0)
    out_copy(_TOPW - 2, out_a, 0).wait()
    out_copy(_TOPW - 1, out_b, 1).wait()


_TCB = 256


def _tc_bottom_body(top_ref, base_ref, out_ref):
    out_ref[...] = base_ref[...]


def _tc_fill_bottom(top, base):
    nblk = (_M - _L) // _TCB
    return pl.pallas_call(
        _tc_bottom_body,
        grid=(nblk,),
        in_specs=[
            pl.BlockSpec(memory_space=pl.ANY),
            pl.BlockSpec((_TCB, _L), lambda i: (i + _L // _TCB, 0)),
        ],
        out_specs=pl.BlockSpec((_TCB, _L), lambda i: (i + _L // _TCB, 0)),
        out_shape=jax.ShapeDtypeStruct((_M, _L), jnp.float32),
        input_output_aliases={0: 0},
    )(top, base)


def kernel(correction, base_orography):
    sc_call = pl.kernel(
        _sc_body,
        out_type=jax.ShapeDtypeStruct((_M, _L), jnp.float32),
        mesh=plsc.VectorSubcoreMesh(core_axis_name="c", subcore_axis_name="s"),
        scratch_types=[
            pltpu.VMEM((_WPAD,), jnp.float32),
            pltpu.VMEM((_WPAD,), jnp.float32),
            pltpu.VMEM((_L,), jnp.float32),
            pltpu.VMEM((_L,), jnp.float32),
            pltpu.VMEM((_L,), jnp.float32),
            pltpu.VMEM((_L,), jnp.float32),
            pltpu.SemaphoreType.DMA((2,)),
            pltpu.SemaphoreType.DMA((2,)),
            pltpu.SemaphoreType.DMA((2,)),
        ],
    )
    top = sc_call(correction, base_orography)
    return _tc_fill_bottom(top, base_orography)


# speedup vs baseline: 1.3267x; 1.3267x over previous
import jax
import jax.numpy as jnp
from jax import lax
from jax.experimental import pallas as pl
from jax.experimental.pallas import tpu as pltpu
from jax.experimental.pallas import tpu_sc as plsc

_M, _L = 8192, 2048
_SCALE = 0.1
_N = (_L * (_L + 1)) // 2
_NW = 32
_WPAD = _L + 128
_TOPW = _L // _NW
_NSET = 4
_BC = 8
_NCH = (_M - _L) // _NW // _BC
_CHUNKS = _L // 16
_UNROLL = 8


def _sc_body(corr, base, out,
             win0, win1, win2, win3,
             bas0, bas1, bas2, bas3,
             out0, out1, out2, out3,
             botb, wsem, bsem, osem, bisem, bosem):
    wid = lax.axis_index("c") * 16 + lax.axis_index("s")
    wins = (win0, win1, win2, win3)
    bases = (bas0, bas1, bas2, bas3)
    outs = (out0, out1, out2, out3)

    def row_params(k):
        m = wid + k * _NW
        start = m * _L - (m * (m + 1)) // 2
        start8 = jnp.minimum((start // 8) * 8, _N - _WPAD)
        start8 = pl.multiple_of(start8, 8)
        return m, start8, start - start8

    def win_copy(k, s):
        _, start8, _ = row_params(k)
        return pltpu.make_async_copy(
            corr.at[pl.ds(start8, _WPAD)], wins[s], wsem.at[s])

    def base_copy(k, s):
        m, _, _ = row_params(k)
        return pltpu.make_async_copy(base.at[m], bases[s], bsem.at[s])

    def out_copy(k, s):
        m, _, _ = row_params(k)
        return pltpu.make_async_copy(outs[s], out.at[m], osem.at[s])

    def compute_row(k, s):
        wbuf, bbuf, obuf = wins[s], bases[s], outs[s]
        m, _, d = row_params(k)
        lane = lax.broadcasted_iota(jnp.int32, (16,), 0)
        dr = d & 15
        dq16 = pl.multiple_of(d - dr, 16)
        ilo = (dr + lane) & 15
        from_lo = (dr + lane) < 16

        def window(j):
            c0 = pl.multiple_of(j * 16, 16)
            off = pl.multiple_of(dq16 + c0, 16)
            off_hi = pl.multiple_of(jnp.minimum(off + 16, _WPAD - 16), 16)
            x_lo = wbuf[pl.ds(off, 16)]
            x_hi = wbuf[pl.ds(off_hi, 16)]
            return jnp.where(
                from_lo,
                x_lo.at[ilo].get(mode="promise_in_bounds"),
                x_hi.at[ilo].get(mode="promise_in_bounds"),
            )

        def copy1(j):
            c0 = pl.multiple_of(j * 16, 16)
            obuf[pl.ds(c0, 16)] = bbuf[pl.ds(c0, 16)]

        def fma1(j):
            c0 = pl.multiple_of(j * 16, 16)
            obuf[pl.ds(c0, 16)] = bbuf[pl.ds(c0, 16)] + _SCALE * window(j)

        jb = m >> 4
        jb8 = jb >> 3
        ju = jb + 1
        ju8 = ((ju + 7) >> 3) << 3

        def copy8(g, c):
            for t in range(_UNROLL):
                copy1(g * _UNROLL + t)
            return c

        def copy_tail(j, c):
            copy1(j)
            return c

        def fma_head(j, c):
            fma1(j)
            return c

        def fma8(g, c):
            for t in range(_UNROLL):
                fma1(g * _UNROLL + t)
            return c

        lax.fori_loop(0, jb8, copy8, 0)
        lax.fori_loop(jb8 * _UNROLL, jb, copy_tail, 0)

        c0 = pl.multiple_of(jb * 16, 16)
        keep = (c0 + lane) >= m
        obuf[pl.ds(c0, 16)] = bbuf[pl.ds(c0, 16)] + jnp.where(
            keep, _SCALE * window(jb), 0.0)

        lax.fori_loop(ju, ju8, fma_head, 0)
        lax.fori_loop(ju8 >> 3, _CHUNKS // _UNROLL, fma8, 0)

    bot0 = _L + wid * _NCH * _BC

    def bin_copy(c, s):
        return pltpu.make_async_copy(
            base.at[pl.ds(bot0 + c * _BC, _BC)], botb.at[s], bisem.at[s])

    def bout_copy(c, s):
        return pltpu.make_async_copy(
            botb.at[s], out.at[pl.ds(bot0 + c * _BC, _BC)], bosem.at[s])

    def bot_step(c):
        s = c & 3

        @pl.when(c >= 2)
        def _drain():
            bout_copy(c - 2, (c - 2) & 3).wait()

        @pl.when(c + 2 < _NCH)
        def _prefetch():
            bin_copy(c + 2, (c + 2) & 3).start()

        bin_copy(c, s).wait()
        bout_copy(c, s).start()

    bin_copy(0, 0).start()
    bin_copy(1, 1).start()
    for s in range(_NSET - 1):
        win_copy(s, s).start()
        base_copy(s, s).start()

    def top_body(k2, carry):
        c1 = (3 * k2) >> 1
        bot_step(c1)

        @pl.when((k2 & 1) == 1)
        def _bot2():
            bot_step(c1 + 1)

        k0 = _NSET * k2
        for t in range(_NSET):
            k = k0 + t

            @pl.when(k + _NSET - 1 < _TOPW)
            def _prefetch():
                win_copy(k + _NSET - 1, (t + _NSET - 1) % _NSET).start()
                base_copy(k + _NSET - 1, (t + _NSET - 1) % _NSET).start()

            win_copy(k, t).wait()
            base_copy(k, t).wait()

            @pl.when(k >= _NSET)
            def _drain():
                out_copy(k - _NSET, t).wait()

            compute_row(k, t)
            out_copy(k, t).start()
        return carry

    lax.fori_loop(0, _TOPW // _NSET, top_body, 0)
    for t in range(_NSET):
        out_copy(_TOPW - _NSET + t, t).wait()
    bout_copy(_NCH - 2, (_NCH - 2) & 3).wait()
    bout_copy(_NCH - 1, (_NCH - 1) & 3).wait()


def kernel(correction, base_orography):
    sc_call = pl.kernel(
        _sc_body,
        out_type=jax.ShapeDtypeStruct((_M, _L), jnp.float32),
        mesh=plsc.VectorSubcoreMesh(core_axis_name="c", subcore_axis_name="s"),
        scratch_types=(
            [pltpu.VMEM((_WPAD,), jnp.float32) for _ in range(4)]
            + [pltpu.VMEM((_L,), jnp.float32) for _ in range(8)]
            + [
                pltpu.VMEM((4, _BC, _L), jnp.float32),
                pltpu.SemaphoreType.DMA((4,)),
                pltpu.SemaphoreType.DMA((4,)),
                pltpu.SemaphoreType.DMA((4,)),
                pltpu.SemaphoreType.DMA((4,)),
                pltpu.SemaphoreType.DMA((4,)),
            ]
        ),
    )
    return sc_call(correction, base_orography)
